# baseline (speedup 1.0000x reference)
import jax
import jax.numpy as jnp
from jax.experimental import pallas as pl
from jax.experimental.pallas import tpu as pltpu

HIDDEN = 128
SEQ = 33
N = 10000

BLOCK = 125
GRID = N // (4 * BLOCK)


def _copy4_kernel(x0, x1, x2, x3, o_ref):
    o_ref[0 * BLOCK:1 * BLOCK] = x0[...]
    o_ref[1 * BLOCK:2 * BLOCK] = x1[...]
    o_ref[2 * BLOCK:3 * BLOCK] = x2[...]
    o_ref[3 * BLOCK:4 * BLOCK] = x3[...]


def kernel(input_embed, token_type_table, ln_weight, ln_bias):
    ispec = lambda k: pl.BlockSpec((BLOCK, SEQ, HIDDEN), lambda i, k=k: (4 * i + k, 0, 0))
    return pl.pallas_call(
        _copy4_kernel,
        grid=(GRID,),
        in_specs=[ispec(0), ispec(1), ispec(2), ispec(3)],
        out_specs=pl.BlockSpec((4 * BLOCK, SEQ, HIDDEN), lambda i: (i, 0, 0)),
        out_shape=jax.ShapeDtypeStruct((N, SEQ, HIDDEN), jnp.float32),
    )(input_embed, input_embed, input_embed, input_embed)


# speedup vs baseline: 1.0154x; 1.0154x over previous
import jax
import jax.numpy as jnp
from jax.experimental import pallas as pl
from jax.experimental.pallas import tpu as pltpu

HIDDEN = 128
SEQ = 33
EPS = 1e-12
N = 10000

B = 125
STEPS = N // B
D = 8


def _compute(x, tt, w, b):
    pos = jax.lax.broadcasted_iota(jnp.int32, (SEQ, 1), 0)
    tte = jnp.where(pos == 0, tt[0][None, :], tt[1][None, :])
    emb = x + tte[None, :, :]
    mean = jnp.mean(emb, axis=-1, keepdims=True)
    cen = emb - mean
    var = jnp.mean(cen * cen, axis=-1, keepdims=True)
    normed = cen * jax.lax.rsqrt(var + EPS)
    return normed * w + b


def _body(x_hbm, tt_ref, w_ref, b_ref, o_hbm, in_buf, out_buf, in_sem, out_sem):
    i = pl.program_id(0)
    slot = i % D

    def in_copy(step, s):
        return pltpu.make_async_copy(
            x_hbm.at[pl.ds(step * B, B)], in_buf.at[s], in_sem.at[s])

    def out_copy(step, s):
        return pltpu.make_async_copy(
            out_buf.at[s], o_hbm.at[pl.ds(step * B, B)], out_sem.at[s])

    @pl.when(i == 0)
    def _():
        for d in range(D):
            in_copy(d, d).start()

    in_copy(i, slot).wait()

    @pl.when(i >= D)
    def _():
        out_copy(i - D, slot).wait()

    out_buf[slot] = _compute(in_buf[slot], tt_ref[...], w_ref[...], b_ref[...])
    out_copy(i, slot).start()

    @pl.when(i + D < STEPS)
    def _():
        in_copy(i + D, slot).start()

    @pl.when(i == STEPS - 1)
    def _():
        for d in range(D):
            out_copy(STEPS - D + d, d).wait()


def kernel(input_embed, token_type_table, ln_weight, ln_bias):
    return pl.pallas_call(
        _body,
        grid=(STEPS,),
        in_specs=[
            pl.BlockSpec(memory_space=pl.ANY),
            pl.BlockSpec((2, HIDDEN), lambda i: (0, 0)),
            pl.BlockSpec((HIDDEN,), lambda i: (0,)),
            pl.BlockSpec((HIDDEN,), lambda i: (0,)),
        ],
        out_specs=pl.BlockSpec(memory_space=pl.ANY),
        out_shape=jax.ShapeDtypeStruct((N, SEQ, HIDDEN), jnp.float32),
        scratch_shapes=[
            pltpu.VMEM((D, B, SEQ, HIDDEN), jnp.float32),
            pltpu.VMEM((D, B, SEQ, HIDDEN), jnp.float32),
            pltpu.SemaphoreType.DMA((D,)),
            pltpu.SemaphoreType.DMA((D,)),
        ],
    )(input_embed, token_type_table, ln_weight, ln_bias)
